# speedup vs baseline: 8.6327x; 8.6327x over previous
import functools

import jax
import jax.numpy as jnp
from jax import lax
from jax.experimental import pallas as pl
from jax.experimental.pallas import tpu as pltpu
from jax.experimental.pallas import tpu_sc as plsc

B = 16384
FEAT_DIM = 256
H = 128
L = 50

NC = 2
NS = 16
NW = NC * NS
B_PER_W = B // NW
CH = 8
N_CHUNKS = B_PER_W // CH

_MESH = plsc.VectorSubcoreMesh(
    core_axis_name="c", subcore_axis_name="s", num_cores=NC, num_subcores=NS
)


def _sc_gather_body(ids_hbm, text_hbm, id_table_hbm, text_table_hbm,
                    pid_hbm, psum_hbm,
                    tidx_v, rows_v, idx_v, idrows_v, acc_v, sem):
    wid = lax.axis_index("s") * NC + lax.axis_index("c")
    base = wid * B_PER_W

    @pl.loop(0, N_CHUNKS)
    def _(c):
        row0 = base + c * CH

        pltpu.sync_copy(ids_hbm.at[pl.ds(row0, CH)], idx_v)
        id_gather = pltpu.async_copy(id_table_hbm.at[idx_v], idrows_v, sem)

        pltpu.sync_copy(text_hbm.at[pl.ds(row0 * L, CH * L)], tidx_v)
        pltpu.async_copy(text_table_hbm.at[tidx_v], rows_v, sem).wait()
        id_gather.wait()

        @pl.loop(0, CH)
        def _(e):
            def add_row(l, accs):
                r = e * L + l
                return tuple(
                    accs[h] + rows_v[r, pl.ds(h * 16, 16)] for h in range(8)
                )

            accs = lax.fori_loop(
                0, L, add_row,
                tuple(jnp.zeros((16,), jnp.float32) for _ in range(8)),
            )
            for h in range(8):
                acc_v[e, pl.ds(h * 16, 16)] = accs[h]

        pltpu.sync_copy(acc_v, psum_hbm.at[pl.ds(row0, CH)])
        pltpu.sync_copy(idrows_v, pid_hbm.at[pl.ds(row0, CH)])


@jax.jit
def _sc_gather(ids, text_flat, id_table, text_table):
    out_type = (
        jax.ShapeDtypeStruct((B, H), jnp.float32),
        jax.ShapeDtypeStruct((B, H), jnp.float32),
    )
    scratch = [
        pltpu.VMEM((CH * L,), jnp.int32),
        pltpu.VMEM((CH * L, H), jnp.float32),
        pltpu.VMEM((CH,), jnp.int32),
        pltpu.VMEM((CH, H), jnp.float32),
        pltpu.VMEM((CH, H), jnp.float32),
        pltpu.SemaphoreType.DMA,
    ]
    return pl.kernel(
        _sc_gather_body, out_type=out_type, mesh=_MESH, scratch_types=scratch
    )(ids, text_flat, id_table, text_table)


BLK = 1024


def _tc_body(feat_ref, w_ref, b_ref, pid_ref, psum_ref, len_ref, out_ref):
    acc = lax.dot_general(
        feat_ref[...], w_ref[...], (((1,), (1,)), ((), ())),
        preferred_element_type=jnp.float32,
    )
    recip = 1.0 / len_ref[...]
    out_ref[...] = acc + b_ref[...] + pid_ref[...] + psum_ref[...] * recip


@jax.jit
def _tc_combine(feat, W, b2, pid, psum, len2):
    grid = (B // BLK,)
    return pl.pallas_call(
        _tc_body,
        grid=grid,
        in_specs=[
            pl.BlockSpec((BLK, FEAT_DIM), lambda i: (i, 0)),
            pl.BlockSpec((H, FEAT_DIM), lambda i: (0, 0)),
            pl.BlockSpec((1, H), lambda i: (0, 0)),
            pl.BlockSpec((BLK, H), lambda i: (i, 0)),
            pl.BlockSpec((BLK, H), lambda i: (i, 0)),
            pl.BlockSpec((BLK, 1), lambda i: (i, 0)),
        ],
        out_specs=pl.BlockSpec((BLK, H), lambda i: (i, 0)),
        out_shape=jax.ShapeDtypeStruct((B, H), jnp.float32),
    )(feat, W, b2, pid, psum, len2)


def kernel(feat, ids, text, text_len, W, b, id_table, text_table):
    ids = ids.astype(jnp.int32)
    text_flat = text.astype(jnp.int32).reshape(B * L)
    pid, psum = _sc_gather(ids, text_flat, id_table, text_table)
    b2 = b.reshape(1, H)
    len2 = text_len.astype(jnp.float32).reshape(B, 1)
    return _tc_combine(feat, W, b2, pid, psum, len2)


# speedup vs baseline: 16.0839x; 1.8631x over previous
import functools

import jax
import jax.numpy as jnp
from jax import lax
from jax.experimental import pallas as pl
from jax.experimental.pallas import tpu as pltpu
from jax.experimental.pallas import tpu_sc as plsc

B = 16384
FEAT_DIM = 256
H = 128
L = 50

NC = 2
NS = 16
NW = NC * NS
B_PER_W = B // NW
CH = 8
N_CHUNKS = B_PER_W // CH
IDC = 128
N_IDC = B_PER_W // IDC

_MESH = plsc.VectorSubcoreMesh(
    core_axis_name="c", subcore_axis_name="s", num_cores=NC, num_subcores=NS
)


def _sc_gather_body(ids_hbm, text_hbm, id_table_hbm, text_table_hbm,
                    pid_hbm, psum_hbm,
                    tidx_v, rows_v, acc_v, iidx_v, idrows_v, sems):
    (sem_g0, sem_g1, sem_x0, sem_x1, sem_o0, sem_o1) = sems
    sem_g = (sem_g0, sem_g1)
    sem_x = (sem_x0, sem_x1)
    sem_o = (sem_o0, sem_o1)

    wid = lax.axis_index("s") * NC + lax.axis_index("c")
    base = wid * B_PER_W

    for b in range(2):
        pltpu.sync_copy(text_hbm.at[pl.ds((base + b * CH) * L, CH * L)],
                        tidx_v[b])
        pltpu.async_copy(text_table_hbm.at[tidx_v[b]], rows_v[b], sem_g[b])

    @pl.loop(0, N_CHUNKS, step=2)
    def _(c):
        for b in range(2):
            c2 = c + b
            pltpu.make_async_copy(
                text_table_hbm.at[tidx_v[b]], rows_v[b], sem_g[b]).wait()

            @pl.when(c2 + 2 < N_CHUNKS)
            def _():
                pltpu.async_copy(
                    text_hbm.at[pl.ds((base + (c2 + 2) * CH) * L, CH * L)],
                    tidx_v[b], sem_x[b])

            @pl.when(c2 >= 2)
            def _():
                pltpu.make_async_copy(
                    acc_v[b], psum_hbm.at[pl.ds(base, CH)], sem_o[b]).wait()

            @pl.loop(0, CH)
            def _(e):
                def add_row(l, accs):
                    r = e * L + l
                    return tuple(
                        accs[h] + rows_v[b][r, pl.ds(h * 16, 16)]
                        for h in range(8)
                    )

                accs = lax.fori_loop(
                    0, L, add_row,
                    tuple(jnp.zeros((16,), jnp.float32) for _ in range(8)),
                )
                for h in range(8):
                    acc_v[b][e, pl.ds(h * 16, 16)] = accs[h]

            pltpu.async_copy(
                acc_v[b], psum_hbm.at[pl.ds(base + c2 * CH, CH)], sem_o[b])

            @pl.when(c2 + 2 < N_CHUNKS)
            def _():
                pltpu.make_async_copy(
                    text_hbm.at[pl.ds((base + (c2 + 2) * CH) * L, CH * L)],
                    tidx_v[b], sem_x[b]).wait()
                pltpu.async_copy(
                    text_table_hbm.at[tidx_v[b]], rows_v[b], sem_g[b])

    for b in range(2):
        pltpu.make_async_copy(
            acc_v[b], psum_hbm.at[pl.ds(base, CH)], sem_o[b]).wait()

    @pl.loop(0, N_IDC)
    def _(k):
        row0 = base + k * IDC
        pltpu.sync_copy(ids_hbm.at[pl.ds(row0, IDC)], iidx_v)
        pltpu.async_copy(id_table_hbm.at[iidx_v], idrows_v, sem_g0).wait()
        pltpu.sync_copy(idrows_v, pid_hbm.at[pl.ds(row0, IDC)])


@jax.jit
def _sc_gather(ids, text_flat, id_table, text_table):
    out_type = (
        jax.ShapeDtypeStruct((B, H), jnp.float32),
        jax.ShapeDtypeStruct((B, H), jnp.float32),
    )
    scratch = [
        (pltpu.VMEM((CH * L,), jnp.int32),) * 2,
        (pltpu.VMEM((CH * L, H), jnp.float32),) * 2,
        (pltpu.VMEM((CH, H), jnp.float32),) * 2,
        pltpu.VMEM((IDC,), jnp.int32),
        pltpu.VMEM((IDC, H), jnp.float32),
        (pltpu.SemaphoreType.DMA,) * 6,
    ]
    return pl.kernel(
        _sc_gather_body, out_type=out_type, mesh=_MESH, scratch_types=scratch
    )(ids, text_flat, id_table, text_table)


BLK = 1024


def _tc_body(feat_ref, w_ref, b_ref, pid_ref, psum_ref, len_ref, out_ref):
    acc = lax.dot_general(
        feat_ref[...], w_ref[...], (((1,), (1,)), ((), ())),
        preferred_element_type=jnp.float32,
    )
    recip = 1.0 / len_ref[...]
    out_ref[...] = acc + b_ref[...] + pid_ref[...] + psum_ref[...] * recip


@jax.jit
def _tc_combine(feat, W, b2, pid, psum, len2):
    grid = (B // BLK,)
    return pl.pallas_call(
        _tc_body,
        grid=grid,
        in_specs=[
            pl.BlockSpec((BLK, FEAT_DIM), lambda i: (i, 0)),
            pl.BlockSpec((H, FEAT_DIM), lambda i: (0, 0)),
            pl.BlockSpec((1, H), lambda i: (0, 0)),
            pl.BlockSpec((BLK, H), lambda i: (i, 0)),
            pl.BlockSpec((BLK, H), lambda i: (i, 0)),
            pl.BlockSpec((BLK, 1), lambda i: (i, 0)),
        ],
        out_specs=pl.BlockSpec((BLK, H), lambda i: (i, 0)),
        out_shape=jax.ShapeDtypeStruct((B, H), jnp.float32),
    )(feat, W, b2, pid, psum, len2)


def kernel(feat, ids, text, text_len, W, b, id_table, text_table):
    ids = ids.astype(jnp.int32)
    text_flat = text.astype(jnp.int32).reshape(B * L)
    pid, psum = _sc_gather(ids, text_flat, id_table, text_table)
    b2 = b.reshape(1, H)
    len2 = text_len.astype(jnp.float32).reshape(B, 1)
    return _tc_combine(feat, W, b2, pid, psum, len2)
